# baseline (speedup 1.0000x reference)
import functools

import jax
import jax.numpy as jnp
from jax import lax
from jax.experimental import pallas as pl
from jax.experimental.pallas import tpu as pltpu
from jax.experimental.pallas import tpu_sc as plsc

_N = 10000
_E = 160000
_D = 128
_NC = 2
_NS = 16
_NW = _NC * _NS
_K = 32
_NBLK = _E // _K
_MAXB = -(-_NBLK // _NW)
_NBUF = 3
_STEPS = _MAXB + _NBUF - 1
_NPAD = 10112
_RPT = _NPAD // _NS
_ZCH = 32
_VL = 30720
_VPT = _VL // _NS
_ZVB = 640

_L16 = (16,)


def _rsqrt(x):
    i = plsc.bitcast(x, jnp.int32)
    y = plsc.bitcast(jnp.int32(0x5F3759DF) - (i >> 1), jnp.float32)
    for _ in range(3):
        y = y * (1.5 - 0.5 * x * y * y)
    return y


def _make_sc_kernel():
    mesh = plsc.VectorSubcoreMesh(core_axis_name="c", subcore_axis_name="s")

    @functools.partial(
        pl.kernel,
        out_type=[
            jax.ShapeDtypeStruct((_NC, _NPAD, _D), jnp.float32),
            jax.ShapeDtypeStruct((_NC * _VL,), jnp.float32),
        ],
        mesh=mesh,
        compiler_params=pltpu.CompilerParams(needs_layout_passes=False),
        scratch_types=[
            pltpu.VMEM((_NBUF, _K), jnp.int32),
            pltpu.VMEM((_NBUF, _K, _D), jnp.float32),
            pltpu.VMEM((_NBUF, _K, _D), jnp.float32),
            pltpu.VMEM((_NBUF, 3 * _K), jnp.float32),
            pltpu.VMEM((_NBUF, 3 * _K), jnp.float32),
            pltpu.VMEM((_NBUF * 3, _K), jnp.int32),
            pltpu.VMEM((_K,), jnp.float32),
            pltpu.VMEM((_ZVB,), jnp.float32),
            pltpu.VMEM_SHARED((_NPAD, _D), jnp.float32),
            pltpu.VMEM_SHARED((_VL,), jnp.float32),
            pltpu.SemaphoreType.DMA((_NBUF,)),
            pltpu.SemaphoreType.DMA((_NBUF,)),
            pltpu.SemaphoreType.DMA((_NBUF,)),
            pltpu.SemaphoreType.DMA((_NBUF,)),
        ],
    )
    def sc_kernel(feat_hbm, idx_hbm, fij_hbm, rb_hbm, wpart, vpart,
                  idx_v, w_v, g_v, r_v, vec_v, idx2_v, s_v, zf_v,
                  w_acc, v_acc, isem, wsem, gsem, ssem):
        c = lax.axis_index("c")
        s = lax.axis_index("s")
        wid = s * _NC + c
        zf = jnp.zeros(_L16, jnp.float32)
        lanes = lax.iota(jnp.int32, 16)
        m15 = lanes == 15
        ci0 = jnp.zeros(_L16, jnp.int32)

        def _zrow(r, carry):
            for cc in range(_D // 16):
                w_v[0, r, pl.ds(cc * 16, 16)] = zf
            return carry
        lax.fori_loop(0, _K, _zrow, None)

        def _zflat(i, carry):
            zf_v[pl.ds(i * 16, 16)] = zf
            return carry
        lax.fori_loop(0, _ZVB // 16, _zflat, None)

        base = s * _RPT
        off = 0
        while off < _RPT:
            ln = min(_ZCH, _RPT - off)
            pltpu.sync_copy(w_v.at[0, pl.ds(0, ln)],
                            w_acc.at[pl.ds(base + off, ln)])
            off += ln
        vbase = s * _VPT
        off = 0
        while off < _VPT:
            ln = min(_ZVB, _VPT - off)
            pltpu.sync_copy(zf_v.at[pl.ds(0, ln)],
                            v_acc.at[pl.ds(vbase + off, ln)])
            off += ln
        plsc.subcore_barrier()

        def scatter_descs(k):
            return [pltpu.make_async_copy(
                        w_v.at[k], w_acc.at[idx_v.at[k]], ssem.at[k])] + [
                    pltpu.make_async_copy(
                        vec_v.at[k, pl.ds(cc * _K, _K)],
                        v_acc.at[idx2_v.at[k * 3 + cc]], ssem.at[k])
                    for cc in range(3)]

        def drain_scatters(t, k):
            b = wid + t * _NW

            @pl.when((t >= 0) & (t < _MAXB) & (b < _NBLK))
            def _():
                for d in scatter_descs(k):
                    d.wait()

        def issue_inputs(t, k):
            b = wid + t * _NW

            @pl.when((t >= 0) & (t < _MAXB) & (b < _NBLK))
            def _():
                e0 = b * _K
                pltpu.async_copy(idx_hbm.at[pl.ds(_E + e0, _K)], idx_v.at[k],
                                 isem.at[k])
                pltpu.async_copy(fij_hbm.at[pl.ds(e0, _K)], w_v.at[k],
                                 wsem.at[k])
                pltpu.async_copy(rb_hbm.at[pl.ds(3 * e0, 3 * _K)], r_v.at[k],
                                 wsem.at[k])

        def issue_gather(t, k):
            b = wid + t * _NW

            @pl.when((t >= 0) & (t < _MAXB) & (b < _NBLK))
            def _():
                pltpu.make_async_copy(idx_hbm.at[pl.ds(_E, _K)], idx_v.at[k],
                                      isem.at[k]).wait()
                pltpu.async_copy(feat_hbm.at[idx_v.at[k]], g_v.at[k],
                                 gsem.at[k])

        def compute_block(t, k):
            b = wid + t * _NW

            @pl.when((t >= 0) & (t < _MAXB) & (b < _NBLK))
            def _():
                pltpu.make_async_copy(fij_hbm.at[pl.ds(0, _K)], w_v.at[k],
                                      wsem.at[k]).wait()
                pltpu.make_async_copy(rb_hbm.at[pl.ds(0, 3 * _K)], r_v.at[k],
                                      wsem.at[k]).wait()
                pltpu.make_async_copy(feat_hbm.at[idx_v.at[k]], g_v.at[k],
                                      gsem.at[k]).wait()

                def edot(i, carry):
                    for kk in range(8):
                        e = i * 8 + kk
                        pr = [w_v[k, e, pl.ds(cg * 16, 16)]
                              * g_v[k, e, pl.ds(cg * 16, 16)]
                              for cg in range(_D // 16)]
                        p = ((pr[0] + pr[1]) + (pr[2] + pr[3])) + (
                            (pr[4] + pr[5]) + (pr[6] + pr[7]))
                        cs = plsc.cumsum(p)
                        plsc.store_scatter(s_v, [ci0 + e], cs, mask=m15)
                    return carry
                lax.fori_loop(0, _K // 8, edot, None)

                for grp in range(_K // 16):
                    j0 = grp * 16
                    sdot = s_v[pl.ds(j0, 16)]
                    ir = (lanes + j0) * 3
                    x = plsc.load_gather(r_v.at[k], [ir])
                    y = plsc.load_gather(r_v.at[k], [ir + 1])
                    z = plsc.load_gather(r_v.at[k], [ir + 2])
                    f = sdot * _rsqrt(x * x + y * y + z * z)
                    vec_v[k, pl.ds(j0, 16)] = x * f
                    vec_v[k, pl.ds(_K + j0, 16)] = y * f
                    vec_v[k, pl.ds(2 * _K + j0, 16)] = z * f
                    idxc = idx_v[k, pl.ds(j0, 16)]
                    for cc in range(3):
                        idx2_v[k * 3 + cc, pl.ds(j0, 16)] = idxc + cc * _NPAD

                pltpu.async_copy(w_v.at[k], w_acc.at[idx_v.at[k]],
                                 ssem.at[k], add=True)
                for cc in range(3):
                    pltpu.async_copy(vec_v.at[k, pl.ds(cc * _K, _K)],
                                     v_acc.at[idx2_v.at[k * 3 + cc]],
                                     ssem.at[k], add=True)

        def pipe(h, carry):
            t0 = h * _NBUF
            for kk in range(_NBUF):
                t = t0 + kk
                drain_scatters(t - _NBUF, kk)
                issue_inputs(t, kk)
                issue_gather(t - 1, (kk - 1) % _NBUF)
                compute_block(t - 2, (kk - 2) % _NBUF)
            return carry
        lax.fori_loop(0, -(-_STEPS // _NBUF), pipe, None)

        for t in range(max(0, _STEPS - _NBUF), _MAXB):
            drain_scatters(t, t % _NBUF)

        plsc.subcore_barrier()
        pltpu.sync_copy(w_acc.at[pl.ds(base, _RPT)],
                        wpart.at[c, pl.ds(base, _RPT)])
        pltpu.sync_copy(v_acc.at[pl.ds(vbase, _VPT)],
                        vpart.at[pl.ds(c * _VL + vbase, _VPT)])

    return sc_kernel


_GRID = 10
_BR = _N // _GRID
_BC = 1024


def _combine_body(feat_ref, wp_ref, vp_ref, rad_ref, nrm_ref):
    wsum = wp_ref[0] + wp_ref[1]
    rad_ref[...] = feat_ref[...] * wsum
    v = vp_ref[0] + vp_ref[1]
    nrm_ref[pl.program_id(0)] = jnp.sqrt(v[0] * v[0] + v[1] * v[1]
                                         + v[2] * v[2])


@functools.cache
def _build():
    sc_kernel = _make_sc_kernel()
    combine = pl.pallas_call(
        _combine_body,
        grid=(_GRID,),
        in_specs=[
            pl.BlockSpec((_BR, _D), lambda g: (g, 0)),
            pl.BlockSpec((_NC, _BR, _D), lambda g: (0, g, 0)),
            pl.BlockSpec((_NC, 3, _BC), lambda g: (0, 0, g)),
        ],
        out_specs=[
            pl.BlockSpec((_BR, _D), lambda g: (g, 0)),
            pl.BlockSpec((_GRID, _BC), lambda g: (0, 0)),
        ],
        out_shape=[
            jax.ShapeDtypeStruct((_N, _D), jnp.float32),
            jax.ShapeDtypeStruct((_GRID, _BC), jnp.float32),
        ],
    )
    return sc_kernel, combine


def kernel(per_atom_feature_tensor, pairlist, f_ij_cutoff, r_ij):
    assert per_atom_feature_tensor.shape == (_N, _D)
    assert f_ij_cutoff.shape == (_E, _D)
    sc_kernel, combine = _build()
    idx2e = pairlist.reshape(2 * _E)
    rflat = r_ij.reshape(3 * _E)
    wpart, vpart = sc_kernel(per_atom_feature_tensor, idx2e, f_ij_cutoff,
                             rflat)
    vpart3 = vpart.reshape(_NC, _VL)[:, :3 * _NPAD].reshape(_NC, 3, _NPAD)
    radial, nrm = combine(per_atom_feature_tensor, wpart, vpart3)
    return radial, nrm.reshape(_GRID * _BC)[:_N]


# speedup vs baseline: 1.0332x; 1.0332x over previous
import functools

import jax
import jax.numpy as jnp
from jax import lax
from jax.experimental import pallas as pl
from jax.experimental.pallas import tpu as pltpu
from jax.experimental.pallas import tpu_sc as plsc

_N = 10000
_E = 160000
_D = 128
_NC = 2
_NS = 16
_NW = _NC * _NS
_K = 32
_NBLK = _E // _K
_MAXB = -(-_NBLK // _NW)
_NBUF = 3
_STEPS = _MAXB + _NBUF - 1
_NPAD = 10112
_RPT = _NPAD // _NS
_ZCH = 32
_VL = 30720
_VPT = _VL // _NS
_ZVB = 640

_L16 = (16,)


def _rsqrt(x):
    i = plsc.bitcast(x, jnp.int32)
    y = plsc.bitcast(jnp.int32(0x5F3759DF) - (i >> 1), jnp.float32)
    for _ in range(3):
        y = y * (1.5 - 0.5 * x * y * y)
    return y


def _make_sc_kernel():
    mesh = plsc.VectorSubcoreMesh(core_axis_name="c", subcore_axis_name="s")

    @functools.partial(
        pl.kernel,
        out_type=[
            jax.ShapeDtypeStruct((_NC, _NPAD, _D), jnp.float32),
            jax.ShapeDtypeStruct((_NC * _VL,), jnp.float32),
        ],
        mesh=mesh,
        compiler_params=pltpu.CompilerParams(needs_layout_passes=False),
        scratch_types=[
            pltpu.VMEM((_NBUF, _K), jnp.int32),
            pltpu.VMEM((_NBUF, _K, _D), jnp.float32),
            pltpu.VMEM((_NBUF, _K, _D), jnp.float32),
            pltpu.VMEM((_NBUF, 3 * _K), jnp.float32),
            pltpu.VMEM((_NBUF, 3 * _K), jnp.float32),
            pltpu.VMEM((_NBUF * 3, _K), jnp.int32),
            pltpu.VMEM((_K,), jnp.float32),
            pltpu.VMEM((_ZVB,), jnp.float32),
            pltpu.VMEM_SHARED((_NPAD, _D), jnp.float32),
            pltpu.VMEM_SHARED((_VL,), jnp.float32),
            pltpu.SemaphoreType.DMA((_NBUF,)),
            pltpu.SemaphoreType.DMA((_NBUF,)),
            pltpu.SemaphoreType.DMA((_NBUF,)),
            pltpu.SemaphoreType.DMA((_NBUF,)),
        ],
    )
    def sc_kernel(feat_hbm, idx_hbm, fij_hbm, rb_hbm, wpart, vpart,
                  idx_v, w_v, g_v, r_v, vec_v, idx2_v, s_v, zf_v,
                  w_acc, v_acc, isem, wsem, gsem, ssem):
        c = lax.axis_index("c")
        s = lax.axis_index("s")
        wid = s * _NC + c
        zf = jnp.zeros(_L16, jnp.float32)
        lanes = lax.iota(jnp.int32, 16)
        m15 = lanes == 15
        ci0 = jnp.zeros(_L16, jnp.int32)

        def _zrow(r, carry):
            for cc in range(_D // 16):
                w_v[0, r, pl.ds(cc * 16, 16)] = zf
            return carry
        lax.fori_loop(0, _K, _zrow, None)

        def _zflat(i, carry):
            zf_v[pl.ds(i * 16, 16)] = zf
            return carry
        lax.fori_loop(0, _ZVB // 16, _zflat, None)

        base = s * _RPT
        off = 0
        while off < _RPT:
            ln = min(_ZCH, _RPT - off)
            pltpu.sync_copy(w_v.at[0, pl.ds(0, ln)],
                            w_acc.at[pl.ds(base + off, ln)])
            off += ln
        vbase = s * _VPT
        off = 0
        while off < _VPT:
            ln = min(_ZVB, _VPT - off)
            pltpu.sync_copy(zf_v.at[pl.ds(0, ln)],
                            v_acc.at[pl.ds(vbase + off, ln)])
            off += ln
        plsc.subcore_barrier()

        def scatter_descs(k):
            return [pltpu.make_async_copy(
                        w_v.at[k], w_acc.at[idx_v.at[k]], ssem.at[k])] + [
                    pltpu.make_async_copy(
                        vec_v.at[k, pl.ds(cc * _K, _K)],
                        v_acc.at[idx2_v.at[k * 3 + cc]], ssem.at[k])
                    for cc in range(3)]

        def drain_scatters(t, k):
            b = wid + t * _NW

            @pl.when((t >= 0) & (t < _MAXB) & (b < _NBLK))
            def _():
                for d in scatter_descs(k):
                    d.wait()

        def issue_inputs(t, k):
            b = wid + t * _NW

            @pl.when((t >= 0) & (t < _MAXB) & (b < _NBLK))
            def _():
                e0 = b * _K
                pltpu.async_copy(idx_hbm.at[pl.ds(_E + e0, _K)], idx_v.at[k],
                                 isem.at[k])
                pltpu.async_copy(fij_hbm.at[pl.ds(e0, _K)], w_v.at[k],
                                 wsem.at[k])
                pltpu.async_copy(rb_hbm.at[pl.ds(3 * e0, 3 * _K)], r_v.at[k],
                                 wsem.at[k])

        def issue_gather(t, k):
            b = wid + t * _NW

            @pl.when((t >= 0) & (t < _MAXB) & (b < _NBLK))
            def _():
                pltpu.make_async_copy(idx_hbm.at[pl.ds(_E, _K)], idx_v.at[k],
                                      isem.at[k]).wait()
                pltpu.async_copy(feat_hbm.at[idx_v.at[k]], g_v.at[k],
                                 gsem.at[k])

        def compute_block(t, k):
            b = wid + t * _NW

            @pl.when((t >= 0) & (t < _MAXB) & (b < _NBLK))
            def _():
                pltpu.make_async_copy(fij_hbm.at[pl.ds(0, _K)], w_v.at[k],
                                      wsem.at[k]).wait()
                pltpu.make_async_copy(rb_hbm.at[pl.ds(0, 3 * _K)], r_v.at[k],
                                      wsem.at[k]).wait()
                pltpu.make_async_copy(feat_hbm.at[idx_v.at[k]], g_v.at[k],
                                      gsem.at[k]).wait()

                def edot(i, carry):
                    for kk in range(4):
                        e = i * 4 + kk
                        p = (w_v[k, e, pl.ds(0, 16)]
                             * g_v[k, e, pl.ds(0, 16)])
                        for cg in range(1, _D // 16):
                            p = p + (w_v[k, e, pl.ds(cg * 16, 16)]
                                     * g_v[k, e, pl.ds(cg * 16, 16)])
                        cs = plsc.cumsum(p)
                        plsc.store_scatter(s_v, [ci0 + e], cs, mask=m15)
                    return carry
                lax.fori_loop(0, _K // 4, edot, None)

                for grp in range(_K // 16):
                    j0 = grp * 16
                    sdot = s_v[pl.ds(j0, 16)]
                    ir = (lanes + j0) * 3
                    x = plsc.load_gather(r_v.at[k], [ir])
                    y = plsc.load_gather(r_v.at[k], [ir + 1])
                    z = plsc.load_gather(r_v.at[k], [ir + 2])
                    f = sdot * _rsqrt(x * x + y * y + z * z)
                    vec_v[k, pl.ds(j0, 16)] = x * f
                    vec_v[k, pl.ds(_K + j0, 16)] = y * f
                    vec_v[k, pl.ds(2 * _K + j0, 16)] = z * f
                    idxc = idx_v[k, pl.ds(j0, 16)]
                    for cc in range(3):
                        idx2_v[k * 3 + cc, pl.ds(j0, 16)] = idxc + cc * _NPAD

                pltpu.async_copy(w_v.at[k], w_acc.at[idx_v.at[k]],
                                 ssem.at[k], add=True)
                for cc in range(3):
                    pltpu.async_copy(vec_v.at[k, pl.ds(cc * _K, _K)],
                                     v_acc.at[idx2_v.at[k * 3 + cc]],
                                     ssem.at[k], add=True)

        def pipe(h, carry):
            t0 = h * _NBUF
            for kk in range(_NBUF):
                t = t0 + kk
                drain_scatters(t - _NBUF, kk)
                issue_inputs(t, kk)
                issue_gather(t - 1, (kk - 1) % _NBUF)
                compute_block(t - 2, (kk - 2) % _NBUF)
            return carry
        lax.fori_loop(0, -(-_STEPS // _NBUF), pipe, None)

        for t in range(max(0, _STEPS - _NBUF), _MAXB):
            drain_scatters(t, t % _NBUF)

        plsc.subcore_barrier()
        pltpu.sync_copy(w_acc.at[pl.ds(base, _RPT)],
                        wpart.at[c, pl.ds(base, _RPT)])
        pltpu.sync_copy(v_acc.at[pl.ds(vbase, _VPT)],
                        vpart.at[pl.ds(c * _VL + vbase, _VPT)])

    return sc_kernel


_GRID = 10
_BR = _N // _GRID
_BC = 1024


def _combine_body(feat_ref, wp_ref, vp_ref, rad_ref, nrm_ref):
    wsum = wp_ref[0] + wp_ref[1]
    rad_ref[...] = feat_ref[...] * wsum
    v = vp_ref[0] + vp_ref[1]
    nrm_ref[pl.program_id(0)] = jnp.sqrt(v[0] * v[0] + v[1] * v[1]
                                         + v[2] * v[2])


@functools.cache
def _build():
    sc_kernel = _make_sc_kernel()
    combine = pl.pallas_call(
        _combine_body,
        grid=(_GRID,),
        in_specs=[
            pl.BlockSpec((_BR, _D), lambda g: (g, 0)),
            pl.BlockSpec((_NC, _BR, _D), lambda g: (0, g, 0)),
            pl.BlockSpec((_NC, 3, _BC), lambda g: (0, 0, g)),
        ],
        out_specs=[
            pl.BlockSpec((_BR, _D), lambda g: (g, 0)),
            pl.BlockSpec((_GRID, _BC), lambda g: (0, 0)),
        ],
        out_shape=[
            jax.ShapeDtypeStruct((_N, _D), jnp.float32),
            jax.ShapeDtypeStruct((_GRID, _BC), jnp.float32),
        ],
    )
    return sc_kernel, combine


def kernel(per_atom_feature_tensor, pairlist, f_ij_cutoff, r_ij):
    assert per_atom_feature_tensor.shape == (_N, _D)
    assert f_ij_cutoff.shape == (_E, _D)
    sc_kernel, combine = _build()
    idx2e = pairlist.reshape(2 * _E)
    rflat = r_ij.reshape(3 * _E)
    wpart, vpart = sc_kernel(per_atom_feature_tensor, idx2e, f_ij_cutoff,
                             rflat)
    vpart3 = vpart.reshape(_NC, _VL)[:, :3 * _NPAD].reshape(_NC, 3, _NPAD)
    radial, nrm = combine(per_atom_feature_tensor, wpart, vpart3)
    return radial, nrm.reshape(_GRID * _BC)[:_N]


# speedup vs baseline: 1.3244x; 1.2819x over previous
import functools

import jax
import jax.numpy as jnp
from jax import lax
from jax.experimental import pallas as pl
from jax.experimental.pallas import tpu as pltpu
from jax.experimental.pallas import tpu_sc as plsc

_N = 10000
_E = 160000
_D = 128
_NC = 2
_NS = 16
_NW = _NC * _NS
_K = 32
_NBLK = _E // _K
_MAXB = -(-_NBLK // _NW)
_NBUF = 3
_STEPS = _MAXB + _NBUF - 1
_NPAD = 10112
_RPT = _NPAD // _NS
_ZCH = 32
_VL = 30720
_VPT = _VL // _NS
_ZVB = 640

_L16 = (16,)


def _rsqrt(x):
    i = plsc.bitcast(x, jnp.int32)
    y = plsc.bitcast(jnp.int32(0x5F3759DF) - (i >> 1), jnp.float32)
    for _ in range(3):
        y = y * (1.5 - 0.5 * x * y * y)
    return y


def _make_sc_kernel():
    mesh = plsc.VectorSubcoreMesh(core_axis_name="c", subcore_axis_name="s")

    @functools.partial(
        pl.kernel,
        out_type=[
            jax.ShapeDtypeStruct((_NC, _NPAD, _D), jnp.float32),
            jax.ShapeDtypeStruct((_NC * _VL,), jnp.float32),
        ],
        mesh=mesh,
        compiler_params=pltpu.CompilerParams(needs_layout_passes=False),
        scratch_types=[
            pltpu.VMEM((_NBUF, _K), jnp.int32),
            pltpu.VMEM((_NBUF, _K, _D), jnp.float32),
            pltpu.VMEM((_NBUF, _K, _D), jnp.float32),
            pltpu.VMEM((_NBUF, 3 * _K), jnp.float32),
            pltpu.VMEM((_NBUF, 3 * _K), jnp.float32),
            pltpu.VMEM((_NBUF * 3, _K), jnp.int32),
            pltpu.VMEM((_K,), jnp.float32),
            pltpu.VMEM((_ZVB,), jnp.float32),
            pltpu.VMEM_SHARED((_NPAD, _D), jnp.float32),
            pltpu.VMEM_SHARED((_VL,), jnp.float32),
            pltpu.SemaphoreType.DMA((_NBUF,)),
            pltpu.SemaphoreType.DMA((_NBUF,)),
            pltpu.SemaphoreType.DMA((_NBUF,)),
            pltpu.SemaphoreType.DMA((_NBUF,)),
        ],
    )
    def sc_kernel(feat_hbm, idx_hbm, fij_hbm, rb_hbm, wpart, vpart,
                  idx_v, w_v, g_v, r_v, vec_v, idx2_v, s_v, zf_v,
                  w_acc, v_acc, isem, wsem, gsem, ssem):
        c = lax.axis_index("c")
        s = lax.axis_index("s")
        wid = s * _NC + c
        zf = jnp.zeros(_L16, jnp.float32)
        lanes = lax.iota(jnp.int32, 16)
        m15 = lanes == 15
        ci0 = jnp.zeros(_L16, jnp.int32)

        def _zrow(r, carry):
            for cc in range(_D // 16):
                w_v[0, r, pl.ds(cc * 16, 16)] = zf
            return carry
        lax.fori_loop(0, _K, _zrow, None)

        def _zflat(i, carry):
            zf_v[pl.ds(i * 16, 16)] = zf
            return carry
        lax.fori_loop(0, _ZVB // 16, _zflat, None)

        base = s * _RPT
        off = 0
        while off < _RPT:
            ln = min(_ZCH, _RPT - off)
            pltpu.sync_copy(w_v.at[0, pl.ds(0, ln)],
                            w_acc.at[pl.ds(base + off, ln)])
            off += ln
        vbase = s * _VPT
        off = 0
        while off < _VPT:
            ln = min(_ZVB, _VPT - off)
            pltpu.sync_copy(zf_v.at[pl.ds(0, ln)],
                            v_acc.at[pl.ds(vbase + off, ln)])
            off += ln
        plsc.subcore_barrier()

        def scatter_descs(k):
            return [pltpu.make_async_copy(
                        w_v.at[k], w_acc.at[idx_v.at[k]], ssem.at[k])] + [
                    pltpu.make_async_copy(
                        vec_v.at[k, pl.ds(cc * _K, _K)],
                        v_acc.at[idx2_v.at[k * 3 + cc]], ssem.at[k])
                    for cc in range(3)]

        def drain_scatters(t, k):
            b = wid + t * _NW

            @pl.when((t >= 0) & (t < _MAXB) & (b < _NBLK))
            def _():
                for d in scatter_descs(k):
                    d.wait()

        def issue_inputs(t, k):
            b = wid + t * _NW

            @pl.when((t >= 0) & (t < _MAXB) & (b < _NBLK))
            def _():
                e0 = b * _K
                pltpu.async_copy(idx_hbm.at[pl.ds(_E + e0, _K)], idx_v.at[k],
                                 isem.at[k])
                pltpu.async_copy(fij_hbm.at[pl.ds(e0, _K)], w_v.at[k],
                                 wsem.at[k])
                pltpu.async_copy(rb_hbm.at[pl.ds(3 * e0, 3 * _K)], r_v.at[k],
                                 wsem.at[k])

        def issue_gather(t, k):
            b = wid + t * _NW

            @pl.when((t >= 0) & (t < _MAXB) & (b < _NBLK))
            def _():
                pltpu.make_async_copy(idx_hbm.at[pl.ds(_E, _K)], idx_v.at[k],
                                      isem.at[k]).wait()
                pltpu.async_copy(feat_hbm.at[idx_v.at[k]], g_v.at[k],
                                 gsem.at[k])

        def compute_block(t, k):
            b = wid + t * _NW

            @pl.when((t >= 0) & (t < _MAXB) & (b < _NBLK))
            def _():
                pltpu.make_async_copy(fij_hbm.at[pl.ds(0, _K)], w_v.at[k],
                                      wsem.at[k]).wait()
                pltpu.make_async_copy(rb_hbm.at[pl.ds(0, 3 * _K)], r_v.at[k],
                                      wsem.at[k]).wait()
                pltpu.make_async_copy(feat_hbm.at[idx_v.at[k]], g_v.at[k],
                                      gsem.at[k]).wait()

                def edot(i, carry):
                    for kk in range(4):
                        e = i * 4 + kk
                        p = (w_v[k, e, pl.ds(0, 16)]
                             * g_v[k, e, pl.ds(0, 16)])
                        for cg in range(1, _D // 16):
                            p = p + (w_v[k, e, pl.ds(cg * 16, 16)]
                                     * g_v[k, e, pl.ds(cg * 16, 16)])
                        cs = plsc.cumsum(p)
                        plsc.store_scatter(s_v, [ci0 + e], cs, mask=m15)
                    return carry
                lax.fori_loop(0, _K // 4, edot, None)

                for grp in range(_K // 16):
                    j0 = grp * 16
                    sdot = s_v[pl.ds(j0, 16)]
                    x = r_v[k, pl.ds(j0, 16)]
                    y = r_v[k, pl.ds(_K + j0, 16)]
                    z = r_v[k, pl.ds(2 * _K + j0, 16)]
                    f = sdot * _rsqrt(x * x + y * y + z * z)
                    vec_v[k, pl.ds(j0, 16)] = x * f
                    vec_v[k, pl.ds(_K + j0, 16)] = y * f
                    vec_v[k, pl.ds(2 * _K + j0, 16)] = z * f
                    idxc = idx_v[k, pl.ds(j0, 16)]
                    for cc in range(3):
                        idx2_v[k * 3 + cc, pl.ds(j0, 16)] = idxc + cc * _NPAD

                pltpu.async_copy(w_v.at[k], w_acc.at[idx_v.at[k]],
                                 ssem.at[k], add=True)
                for cc in range(3):
                    pltpu.async_copy(vec_v.at[k, pl.ds(cc * _K, _K)],
                                     v_acc.at[idx2_v.at[k * 3 + cc]],
                                     ssem.at[k], add=True)

        def pipe(h, carry):
            t0 = h * _NBUF
            for kk in range(_NBUF):
                t = t0 + kk
                drain_scatters(t - _NBUF, kk)
                issue_inputs(t, kk)
                issue_gather(t - 1, (kk - 1) % _NBUF)
                compute_block(t - 2, (kk - 2) % _NBUF)
            return carry
        lax.fori_loop(0, -(-_STEPS // _NBUF), pipe, None)

        for t in range(max(0, _STEPS - _NBUF), _MAXB):
            drain_scatters(t, t % _NBUF)

        plsc.subcore_barrier()
        pltpu.sync_copy(w_acc.at[pl.ds(base, _RPT)],
                        wpart.at[c, pl.ds(base, _RPT)])
        pltpu.sync_copy(v_acc.at[pl.ds(vbase, _VPT)],
                        vpart.at[pl.ds(c * _VL + vbase, _VPT)])

    return sc_kernel


_GRID = 10
_BR = _N // _GRID
_BC = 1024


def _combine_body(feat_ref, wp_ref, vp_ref, rad_ref, nrm_ref):
    wsum = wp_ref[0] + wp_ref[1]
    rad_ref[...] = feat_ref[...] * wsum
    v = vp_ref[0] + vp_ref[1]
    nrm_ref[pl.program_id(0)] = jnp.sqrt(v[0] * v[0] + v[1] * v[1]
                                         + v[2] * v[2])


@functools.cache
def _build():
    sc_kernel = _make_sc_kernel()
    combine = pl.pallas_call(
        _combine_body,
        grid=(_GRID,),
        in_specs=[
            pl.BlockSpec((_BR, _D), lambda g: (g, 0)),
            pl.BlockSpec((_NC, _BR, _D), lambda g: (0, g, 0)),
            pl.BlockSpec((_NC, 3, _BC), lambda g: (0, 0, g)),
        ],
        out_specs=[
            pl.BlockSpec((_BR, _D), lambda g: (g, 0)),
            pl.BlockSpec((_GRID, _BC), lambda g: (0, 0)),
        ],
        out_shape=[
            jax.ShapeDtypeStruct((_N, _D), jnp.float32),
            jax.ShapeDtypeStruct((_GRID, _BC), jnp.float32),
        ],
    )
    return sc_kernel, combine


def kernel(per_atom_feature_tensor, pairlist, f_ij_cutoff, r_ij):
    assert per_atom_feature_tensor.shape == (_N, _D)
    assert f_ij_cutoff.shape == (_E, _D)
    sc_kernel, combine = _build()
    idx2e = pairlist.reshape(2 * _E)
    rb = r_ij.T.reshape(3, _NBLK, _K).transpose(1, 0, 2).reshape(-1)
    wpart, vpart = sc_kernel(per_atom_feature_tensor, idx2e, f_ij_cutoff, rb)
    vpart3 = vpart.reshape(_NC, _VL)[:, :3 * _NPAD].reshape(_NC, 3, _NPAD)
    radial, nrm = combine(per_atom_feature_tensor, wpart, vpart3)
    return radial, nrm.reshape(_GRID * _BC)[:_N]
